# Initial kernel scaffold; baseline (speedup 1.0000x reference)
import jax
import jax.numpy as jnp
from jax.experimental import pallas as pl


def kernel(x, embeddings):
    raise NotImplementedError("write your pallas kernel here")


# speedup vs baseline: 1.8635x; 1.8635x over previous
import functools

import jax
import jax.numpy as jnp
from jax import lax
from jax.experimental import pallas as pl
from jax.experimental.pallas import tpu as pltpu
from jax.experimental.pallas import tpu_sc as plsc

K = 512
D = 256
T = 256
COMMITMENT = 0.25

T_BLK = 8
N_STEPS = T // T_BLK


def _argmin_loss_body(x_blk_ref, x_full_ref, emb_ref, idx_ref, loss_ref,
                      f_ref, acc_ref):
    s = pl.program_id(0)
    emb = emb_ref[...]

    @pl.when(s == 0)
    def _init():
        x_full = x_full_ref[...]
        g = lax.dot_general(emb, x_full, (((1,), (1,)), ((), ())),
                            preferred_element_type=jnp.float32)
        enorm2 = jnp.sum(emb * emb, axis=1, keepdims=True)
        f_ref[...] = enorm2 - 2.0 * g
        acc_ref[0, 0] = float(T) * jnp.sum(x_full * x_full)

    xblk = x_blk_ref[...]
    diff = xblk[:, None, :] - emb[None, :, :]
    d = diff * diff
    m = jnp.min(d, axis=1)
    iota_k = lax.broadcasted_iota(jnp.int32, (T_BLK, K, D), 1)
    hit = d == m[:, None, :]
    idx = jnp.min(jnp.where(hit, iota_k, K), axis=1)
    idx_ref[...] = idx

    onehot = iota_k == idx[:, None, :]
    f = f_ref[...]
    contrib = jnp.sum(jnp.where(onehot, f[None, :, :], 0.0))
    acc_ref[0, 0] += contrib

    @pl.when(s == N_STEPS - 1)
    def _fin():
        loss_ref[0, 0] = acc_ref[0, 0] * ((1.0 + COMMITMENT) / (T * D * D))


def _argmin_loss_call(x2d, emb):
    return pl.pallas_call(
        _argmin_loss_body,
        grid=(N_STEPS,),
        in_specs=[
            pl.BlockSpec((T_BLK, D), lambda s: (s, 0)),
            pl.BlockSpec((T, D), lambda s: (0, 0)),
            pl.BlockSpec((K, D), lambda s: (0, 0)),
        ],
        out_specs=[
            pl.BlockSpec((T_BLK, D), lambda s: (s, 0)),
            pl.BlockSpec(memory_space=pltpu.SMEM, block_shape=(1, 1),
                         index_map=lambda s: (0, 0)),
        ],
        out_shape=[
            jax.ShapeDtypeStruct((T, D), jnp.int32),
            jax.ShapeDtypeStruct((1, 1), jnp.float32),
        ],
        scratch_shapes=[
            pltpu.VMEM((K, D), jnp.float32),
            pltpu.SMEM((1, 1), jnp.float32),
        ],
        compiler_params=pltpu.CompilerParams(
            dimension_semantics=("arbitrary",),
        ),
    )(x2d, x2d, emb)


_NC = 2
_NS = 16
_NW = _NC * _NS
_B = T * D
_ROWS_PER_W = _B // _NW
_CHUNK = 128
_N_CHUNKS = _ROWS_PER_W // _CHUNK


def _sc_gather_body(emb_hbm, idx_hbm, out_hbm, idx_v, rows_v, sem):
    wid = lax.axis_index("s") * _NC + lax.axis_index("c")
    w_base = wid * _ROWS_PER_W

    def chunk(i, carry):
        base = w_base + i * _CHUNK
        pltpu.sync_copy(idx_hbm.at[pl.ds(base, _CHUNK)], idx_v)
        pltpu.async_copy(emb_hbm.at[idx_v], rows_v, sem).wait()
        pltpu.sync_copy(rows_v, out_hbm.at[pl.ds(base, _CHUNK)])
        return carry

    lax.fori_loop(0, _N_CHUNKS, chunk, 0)


@functools.cache
def _sc_gather_kernel():
    return pl.kernel(
        _sc_gather_body,
        out_type=jax.ShapeDtypeStruct((_B, D), jnp.float32),
        mesh=plsc.VectorSubcoreMesh(core_axis_name="c", subcore_axis_name="s",
                                    num_cores=_NC, num_subcores=_NS),
        scratch_types=[
            pltpu.VMEM((_CHUNK,), jnp.int32),
            pltpu.VMEM((_CHUNK, D), jnp.float32),
            pltpu.SemaphoreType.DMA,
        ],
    )


def _sc_gather_call(emb, idx_flat):
    return _sc_gather_kernel()(emb, idx_flat)


def kernel(x, embeddings):
    x2d = x[0]
    idx, loss = _argmin_loss_call(x2d, embeddings)
    quant = _sc_gather_call(embeddings, idx.reshape(_B))
    return quant.reshape(1, T, D, D), loss[0, 0]


# speedup vs baseline: 1.8692x; 1.0031x over previous
import functools

import jax
import jax.numpy as jnp
from jax import lax
from jax.experimental import pallas as pl
from jax.experimental.pallas import tpu as pltpu
from jax.experimental.pallas import tpu_sc as plsc

K = 512
D = 256
T = 256
COMMITMENT = 0.25

T_BLK = 8
N_STEPS = T // T_BLK


def _argmin_loss_body(x_blk_ref, x_full_ref, emb_ref, idx_ref, loss_ref,
                      f_ref, acc_ref):
    s = pl.program_id(0)
    emb = emb_ref[...]

    @pl.when(s == 0)
    def _init():
        x_full = x_full_ref[...]
        g = lax.dot_general(emb, x_full, (((1,), (1,)), ((), ())),
                            preferred_element_type=jnp.float32)
        enorm2 = jnp.sum(emb * emb, axis=1, keepdims=True)
        f_ref[...] = enorm2 - 2.0 * g
        acc_ref[0, 0] = float(T) * jnp.sum(x_full * x_full)

    xblk = x_blk_ref[...]
    diff = xblk[:, None, :] - emb[None, :, :]
    d = diff * diff
    m = jnp.min(d, axis=1)
    iota_k = lax.broadcasted_iota(jnp.int32, (T_BLK, K, D), 1)
    hit = d == m[:, None, :]
    idx = jnp.min(jnp.where(hit, iota_k, K), axis=1)
    idx_ref[...] = idx

    onehot = iota_k == idx[:, None, :]
    f = f_ref[...]
    contrib = jnp.sum(jnp.where(onehot, f[None, :, :], 0.0))
    acc_ref[0, 0] += contrib

    @pl.when(s == N_STEPS - 1)
    def _fin():
        loss_ref[0, 0] = acc_ref[0, 0] * ((1.0 + COMMITMENT) / (T * D * D))


def _argmin_loss_call(x2d, emb):
    return pl.pallas_call(
        _argmin_loss_body,
        grid=(N_STEPS,),
        in_specs=[
            pl.BlockSpec((T_BLK, D), lambda s: (s, 0)),
            pl.BlockSpec((T, D), lambda s: (0, 0)),
            pl.BlockSpec((K, D), lambda s: (0, 0)),
        ],
        out_specs=[
            pl.BlockSpec((T_BLK, D), lambda s: (s, 0)),
            pl.BlockSpec(memory_space=pltpu.SMEM, block_shape=(1, 1),
                         index_map=lambda s: (0, 0)),
        ],
        out_shape=[
            jax.ShapeDtypeStruct((T, D), jnp.int32),
            jax.ShapeDtypeStruct((1, 1), jnp.float32),
        ],
        scratch_shapes=[
            pltpu.VMEM((K, D), jnp.float32),
            pltpu.SMEM((1, 1), jnp.float32),
        ],
        compiler_params=pltpu.CompilerParams(
            dimension_semantics=("arbitrary",),
        ),
    )(x2d, x2d, emb)


_NC = 2
_NS = 16
_NW = _NC * _NS
_B = T * D
_ROWS_PER_W = _B // _NW
_CHUNK = 128
_N_CHUNKS = _ROWS_PER_W // _CHUNK


def _sc_gather_body(emb_hbm, idx_hbm, out_hbm,
                    idx_v0, idx_v1, rows_v0, rows_v1, sem0, sem1):
    wid = lax.axis_index("s") * _NC + lax.axis_index("c")
    w_base = wid * _ROWS_PER_W
    n_pairs = _N_CHUNKS // 2

    def fire(idx_v, rows_v, sem, c):
        pltpu.sync_copy(idx_hbm.at[pl.ds(w_base + c * _CHUNK, _CHUNK)], idx_v)
        pltpu.make_async_copy(emb_hbm.at[idx_v], rows_v, sem).start()

    def drain(idx_v, rows_v, sem, c):
        pltpu.make_async_copy(emb_hbm.at[idx_v], rows_v, sem).wait()
        pltpu.sync_copy(rows_v, out_hbm.at[pl.ds(w_base + c * _CHUNK, _CHUNK)])

    fire(idx_v0, rows_v0, sem0, 0)

    def pair(g, carry):
        c0 = 2 * g
        fire(idx_v1, rows_v1, sem1, c0 + 1)
        drain(idx_v0, rows_v0, sem0, c0)

        @pl.when(g < n_pairs - 1)
        def _():
            fire(idx_v0, rows_v0, sem0, c0 + 2)

        drain(idx_v1, rows_v1, sem1, c0 + 1)
        return carry

    lax.fori_loop(0, n_pairs, pair, 0)


@functools.cache
def _sc_gather_kernel():
    return pl.kernel(
        _sc_gather_body,
        out_type=jax.ShapeDtypeStruct((_B, D), jnp.float32),
        mesh=plsc.VectorSubcoreMesh(core_axis_name="c", subcore_axis_name="s",
                                    num_cores=_NC, num_subcores=_NS),
        scratch_types=[
            pltpu.VMEM((_CHUNK,), jnp.int32),
            pltpu.VMEM((_CHUNK,), jnp.int32),
            pltpu.VMEM((_CHUNK, D), jnp.float32),
            pltpu.VMEM((_CHUNK, D), jnp.float32),
            pltpu.SemaphoreType.DMA,
            pltpu.SemaphoreType.DMA,
        ],
    )


def _sc_gather_call(emb, idx_flat):
    return _sc_gather_kernel()(emb, idx_flat)


def kernel(x, embeddings):
    x2d = x[0]
    idx, loss = _argmin_loss_call(x2d, embeddings)
    quant = _sc_gather_call(embeddings, idx.reshape(_B))
    return quant.reshape(1, T, D, D), loss[0, 0]


# speedup vs baseline: 2.4184x; 1.2938x over previous
import functools

import jax
import jax.numpy as jnp
from jax import lax
from jax.experimental import pallas as pl
from jax.experimental.pallas import tpu as pltpu
from jax.experimental.pallas import tpu_sc as plsc

K = 512
D = 256
T = 256
COMMITMENT = 0.25

N_STAGES = 4
T_STAGE = T // N_STAGES
T_BLK = 8
N_STEPS = T_STAGE // T_BLK

_LOSS_SCALE = (1.0 + COMMITMENT) / (T * D * D)


def _tc_stage_body(x_blk_ref, x_full_ref, emb_ref, idx_ref, loss_ref,
                   f_ref, acc_ref):
    s = pl.program_id(0)
    emb = emb_ref[...]

    @pl.when(s == 0)
    def _init():
        x_full = x_full_ref[...]
        g = lax.dot_general(emb, x_full, (((1,), (1,)), ((), ())),
                            preferred_element_type=jnp.float32)
        enorm2 = jnp.sum(emb * emb, axis=1, keepdims=True)
        f_ref[...] = enorm2 - 2.0 * g
        acc_ref[0, 0] = float(T_STAGE) * jnp.sum(x_full * x_full)

    xblk = x_blk_ref[...]
    diff = xblk[:, None, :] - emb[None, :, :]
    d = diff * diff
    m = jnp.min(d, axis=1)
    iota_k = lax.broadcasted_iota(jnp.int32, (T_BLK, K, D), 1)
    hit = d == m[:, None, :]
    idx = jnp.min(jnp.where(hit, iota_k, K), axis=1)
    idx_ref[...] = idx

    onehot = iota_k == idx[:, None, :]
    f = f_ref[...]
    contrib = jnp.sum(jnp.where(onehot, f[None, :, :], 0.0))
    acc_ref[0, 0] += contrib

    @pl.when(s == N_STEPS - 1)
    def _fin():
        loss_ref[0, 0] = acc_ref[0, 0] * _LOSS_SCALE


def _tc_stage_call(x_stage, x_full, emb):
    return pl.pallas_call(
        _tc_stage_body,
        grid=(N_STEPS,),
        in_specs=[
            pl.BlockSpec((T_BLK, D), lambda s: (s, 0)),
            pl.BlockSpec((T, D), lambda s: (0, 0)),
            pl.BlockSpec((K, D), lambda s: (0, 0)),
        ],
        out_specs=[
            pl.BlockSpec((T_BLK, D), lambda s: (s, 0)),
            pl.BlockSpec(memory_space=pltpu.SMEM, block_shape=(1, 1),
                         index_map=lambda s: (0, 0)),
        ],
        out_shape=[
            jax.ShapeDtypeStruct((T_STAGE, D), jnp.int32),
            jax.ShapeDtypeStruct((1, 1), jnp.float32),
        ],
        scratch_shapes=[
            pltpu.VMEM((K, D), jnp.float32),
            pltpu.SMEM((1, 1), jnp.float32),
        ],
        compiler_params=pltpu.CompilerParams(
            dimension_semantics=("arbitrary",),
        ),
    )(x_stage, x_full, emb)


_NC = 2
_NS = 16
_NW = _NC * _NS
_B = T * D
_B_STAGE = T_STAGE * D
_ROWS_PER_W = _B_STAGE // _NW
_CHUNK = 128
_N_CHUNKS = _ROWS_PER_W // _CHUNK


def _sc_stage_body(stage, emb_hbm, idx_hbm, out_hbm,
                   idx_v0, idx_v1, rows_v0, rows_v1, sem0, sem1):
    wid = lax.axis_index("s") * _NC + lax.axis_index("c")
    w_base = wid * _ROWS_PER_W
    out_base = stage * _B_STAGE + w_base
    n_pairs = _N_CHUNKS // 2

    def fire(idx_v, rows_v, sem, c):
        pltpu.sync_copy(idx_hbm.at[pl.ds(w_base + c * _CHUNK, _CHUNK)], idx_v)
        pltpu.make_async_copy(emb_hbm.at[idx_v], rows_v, sem).start()

    def drain(idx_v, rows_v, sem, c):
        pltpu.make_async_copy(emb_hbm.at[idx_v], rows_v, sem).wait()
        pltpu.sync_copy(rows_v, out_hbm.at[pl.ds(out_base + c * _CHUNK,
                                                 _CHUNK)])

    fire(idx_v0, rows_v0, sem0, 0)

    def pair(g, carry):
        c0 = 2 * g
        fire(idx_v1, rows_v1, sem1, c0 + 1)
        drain(idx_v0, rows_v0, sem0, c0)

        @pl.when(g < n_pairs - 1)
        def _():
            fire(idx_v0, rows_v0, sem0, c0 + 2)

        drain(idx_v1, rows_v1, sem1, c0 + 1)
        return carry

    lax.fori_loop(0, n_pairs, pair, 0)


@functools.cache
def _sc_stage_kernel(stage):
    return pl.kernel(
        functools.partial(_sc_stage_body, stage),
        out_type=(),
        mesh=plsc.VectorSubcoreMesh(core_axis_name="c", subcore_axis_name="s",
                                    num_cores=_NC, num_subcores=_NS),
        scratch_types=[
            pltpu.VMEM((_CHUNK,), jnp.int32),
            pltpu.VMEM((_CHUNK,), jnp.int32),
            pltpu.VMEM((_CHUNK, D), jnp.float32),
            pltpu.VMEM((_CHUNK, D), jnp.float32),
            pltpu.SemaphoreType.DMA,
            pltpu.SemaphoreType.DMA,
        ],
        name=f"sc_gather_stage{stage}",
    )


def kernel(x, embeddings):
    x2d = x[0]
    out_ref = jax.new_ref(pl.empty((_B, D), jnp.float32))
    loss = jnp.float32(0.0)
    for s in range(N_STAGES):
        x_stage = lax.slice_in_dim(x2d, s * T_STAGE, (s + 1) * T_STAGE)
        idx_s, loss_s = _tc_stage_call(x_stage, x2d, embeddings)
        _sc_stage_kernel(s)(embeddings, idx_s.reshape(_B_STAGE), out_ref)
        loss = loss + loss_s[0, 0]
    return out_ref[...].reshape(1, T, D, D), loss
